# baseline (speedup 1.0000x reference)
import jax
import jax.numpy as jnp
from jax.experimental import pallas as pl
from jax.experimental.pallas import tpu as pltpu

W_POINT, W_CHAMFER, W_STRUCTURE = 1.0, 5.0, 2.0
BIG = 1e10


def _loss_kernel(pxr, pyr, pxc, pyc, pvc, txr, tyr, tvr, s0r, s1r, p2m, t2t,
                 out_total, out_point, out_chamfer, acc):
    b = pl.program_id(0)
    nb = pl.num_programs(0)

    @pl.when(b == 0)
    def _init():
        acc[0] = 0.0
        acc[1] = 0.0
        acc[2] = 0.0
        acc[3] = 0.0

    px_r = pxr[0]
    py_r = pyr[0]
    tx_r = txr[0]
    ty_r = tyr[0]
    tv_r = tvr[0]

    v = (tv_r == 1.0).astype(jnp.float32)
    ex = px_r - tx_r
    ey = py_r - ty_r
    err2 = ex * ex + ey * ey
    point_sse = jnp.sum(v * err2)
    smask = jnp.clip(s0r[0] + s1r[0], 0.0, 1.0)
    tmask = smask * v
    struct_sse = jnp.sum(tmask * err2)
    tm_sum = jnp.sum(tmask)

    px_c = pxc[0]
    py_c = pyc[0]
    pv_c = pvc[0]
    pm_c = (pv_c == 1.0).astype(jnp.float32)
    cnt_p = jnp.sum(pm_c)
    cnt_t = jnp.sum(v)

    a2_c = px_c * px_c + py_c * py_c
    b2_r = tx_r * tx_r + ty_r * ty_r
    b2t_r = b2_r + (1.0 - v) * BIG
    a2p_c = a2_c + (1.0 - pm_c) * BIG

    cross = jax.lax.dot_general(
        (-2.0 * p2m[0]).astype(jnp.bfloat16),
        t2t[0].astype(jnp.bfloat16),
        (((1,), (0,)), ((), ())),
        preferred_element_type=jnp.float32)

    rowmin = jnp.min(cross + b2t_r, axis=1, keepdims=True)
    d2row = jnp.maximum(a2_c + rowmin, 0.0) + 1e-12
    min_dist_pred = jnp.sqrt(d2row)
    mean_p = jnp.sum(pm_c * min_dist_pred) / jnp.maximum(cnt_p, 1.0)

    colmin = jnp.min(cross + a2p_c, axis=0, keepdims=True)
    d2col = jnp.maximum(b2_r + colmin, 0.0) + 1e-12
    min_dist_tgt = jnp.sqrt(d2col)
    mean_t = jnp.sum(v * min_dist_tgt) / jnp.maximum(cnt_t, 1.0)

    valid_b = ((cnt_p > 0.0) & (cnt_t > 0.0)).astype(jnp.float32)
    chamfer = valid_b * (mean_p + mean_t) * 0.5

    acc[0] = acc[0] + point_sse
    acc[1] = acc[1] + struct_sse
    acc[2] = acc[2] + tm_sum
    acc[3] = acc[3] + chamfer

    @pl.when(b == nb - 1)
    def _finalize():
        n_elems = jnp.float32(nb) * jnp.float32(2 * pxr.shape[2])
        loss_point = acc[0] / n_elems
        loss_structure = jnp.where(acc[2] == 0.0, 0.0, acc[1] / n_elems)
        loss_chamfer = acc[3] / jnp.float32(nb)
        out_point[0, 0] = loss_point
        out_chamfer[0, 0] = loss_chamfer
        out_total[0, 0] = (W_POINT * loss_point + W_CHAMFER * loss_chamfer
                           + W_STRUCTURE * loss_structure)


def kernel(pred, target, skeleton_mask):
    B, N, _ = pred.shape
    f32 = jnp.float32

    pxr = pred[:, :, 0].reshape(B, 1, N)
    pyr = pred[:, :, 1].reshape(B, 1, N)
    pxc = pred[:, :, 0].reshape(B, N, 1)
    pyc = pred[:, :, 1].reshape(B, N, 1)
    pvc = pred[:, :, 2].reshape(B, N, 1)
    txr = target[:, :, 0].reshape(B, 1, N)
    tyr = target[:, :, 1].reshape(B, 1, N)
    tvr = target[:, :, 2].reshape(B, 1, N)
    s0r = skeleton_mask[:, :, 0].astype(f32).reshape(B, 1, N)
    s1r = skeleton_mask[:, :, 1].astype(f32).reshape(B, 1, N)
    p2m = pred[:, :, :2]
    t2t = jnp.swapaxes(target[:, :, :2], 1, 2)

    row_spec = pl.BlockSpec((1, 1, N), lambda b: (b, 0, 0))
    col_spec = pl.BlockSpec((1, N, 1), lambda b: (b, 0, 0))
    pm_spec = pl.BlockSpec((1, N, 2), lambda b: (b, 0, 0))
    tt_spec = pl.BlockSpec((1, 2, N), lambda b: (b, 0, 0))
    out_spec = pl.BlockSpec(memory_space=pltpu.SMEM)

    out_shape = [jax.ShapeDtypeStruct((1, 1), f32)] * 3
    total, point, chamfer = pl.pallas_call(
        _loss_kernel,
        grid=(B,),
        in_specs=[row_spec, row_spec, col_spec, col_spec, col_spec,
                  row_spec, row_spec, row_spec, row_spec, row_spec,
                  pm_spec, tt_spec],
        out_specs=[out_spec, out_spec, out_spec],
        out_shape=out_shape,
        scratch_shapes=[pltpu.SMEM((4,), f32)],
    )(pxr, pyr, pxc, pyc, pvc, txr, tyr, tvr, s0r, s1r, p2m, t2t)

    return (total[0, 0], point[0, 0], jnp.zeros((), f32), chamfer[0, 0])


# speedup vs baseline: 1.6733x; 1.6733x over previous
import jax
import jax.numpy as jnp
from jax.experimental import pallas as pl
from jax.experimental.pallas import tpu as pltpu

W_POINT, W_CHAMFER, W_STRUCTURE = 1.0, 5.0, 2.0
BIG = 1e10


def _split3_bf16(x):
    hi = x.astype(jnp.bfloat16)
    r1 = x - hi.astype(jnp.float32)
    mid = r1.astype(jnp.bfloat16)
    lo = (r1 - mid.astype(jnp.float32)).astype(jnp.bfloat16)
    return hi, mid, lo


def _loss_kernel(rows, out_total, out_point, out_chamfer):
    B = rows.shape[0]
    N = rows.shape[2]
    bf16 = jnp.bfloat16
    f32 = jnp.float32
    one = jnp.ones((1, N), bf16)
    zero3 = jnp.zeros((3, N), bf16)
    dn = (((0,), (0,)), ((), ()))

    def sample(b, carry):
        pt_acc, st_acc, tm_acc, ch_acc = carry
        blk = rows[b]
        px, py, pv = blk[0:1], blk[1:2], blk[2:3]
        tx, ty, tv = blk[3:4], blk[4:5], blk[5:6]
        s0, s1 = blk[6:7], blk[7:8]

        v = (tv == 1.0).astype(f32)
        ex = px - tx
        ey = py - ty
        err2 = ex * ex + ey * ey
        point_sse = jnp.sum(v * err2)
        smask = jnp.clip(s0 + s1, 0.0, 1.0)
        tmask = smask * v
        struct_sse = jnp.sum(tmask * err2)
        tm_sum = jnp.sum(tmask)

        pm = (pv == 1.0).astype(f32)
        cnt_p = jnp.sum(pm)
        cnt_t = jnp.sum(v)

        a2 = px * px + py * py
        b2 = tx * tx + ty * ty
        b2t = b2 + (1.0 - v) * BIG
        a2p = a2 + (1.0 - pm) * BIG

        pxb = px.astype(bf16)
        pyb = py.astype(bf16)
        txb = tx.astype(bf16)
        tyb = ty.astype(bf16)
        mpx = -2.0 * pxb
        mpy = -2.0 * pyb

        bhi, bmid, blo = _split3_bf16(b2t)
        ahi, amid, alo = _split3_bf16(a2p)

        t_pen = jnp.concatenate([txb, tyb, bhi, bmid, blo, zero3], axis=0)
        p_one = jnp.concatenate([mpx, mpy, one, one, one, zero3], axis=0)
        m2 = jax.lax.dot_general(t_pen, p_one, dn, preferred_element_type=f32)
        predmin = jnp.min(m2, axis=0, keepdims=True)

        p_pen = jnp.concatenate([mpx, mpy, ahi, amid, alo, zero3], axis=0)
        t_one = jnp.concatenate([txb, tyb, one, one, one, zero3], axis=0)
        m1 = jax.lax.dot_general(p_pen, t_one, dn, preferred_element_type=f32)
        tgtmin = jnp.min(m1, axis=0, keepdims=True)

        d2p = jnp.maximum(a2 + predmin, 0.0) + 1e-12
        mean_p = jnp.sum(pm * jnp.sqrt(d2p)) / jnp.maximum(cnt_p, 1.0)
        d2t = jnp.maximum(b2 + tgtmin, 0.0) + 1e-12
        mean_t = jnp.sum(v * jnp.sqrt(d2t)) / jnp.maximum(cnt_t, 1.0)

        valid_b = ((cnt_p > 0.0) & (cnt_t > 0.0)).astype(f32)
        chamfer = valid_b * (mean_p + mean_t) * 0.5

        return (pt_acc + point_sse, st_acc + struct_sse,
                tm_acc + tm_sum, ch_acc + chamfer)

    z = jnp.float32(0.0)
    pt_acc, st_acc, tm_acc, ch_acc = jax.lax.fori_loop(
        0, B, sample, (z, z, z, z))

    n_elems = jnp.float32(B) * jnp.float32(2 * N)
    loss_point = pt_acc / n_elems
    loss_structure = jnp.where(tm_acc == 0.0, 0.0, st_acc / n_elems)
    loss_chamfer = ch_acc / jnp.float32(B)
    out_point[0, 0] = loss_point
    out_chamfer[0, 0] = loss_chamfer
    out_total[0, 0] = (W_POINT * loss_point + W_CHAMFER * loss_chamfer
                       + W_STRUCTURE * loss_structure)


def kernel(pred, target, skeleton_mask):
    B, N, _ = pred.shape
    f32 = jnp.float32

    rows = jnp.stack(
        [pred[:, :, 0], pred[:, :, 1], pred[:, :, 2],
         target[:, :, 0], target[:, :, 1], target[:, :, 2],
         skeleton_mask[:, :, 0].astype(f32),
         skeleton_mask[:, :, 1].astype(f32)], axis=1)

    out_spec = pl.BlockSpec(memory_space=pltpu.SMEM)
    out_shape = [jax.ShapeDtypeStruct((1, 1), f32)] * 3
    total, point, chamfer = pl.pallas_call(
        _loss_kernel,
        out_specs=[out_spec, out_spec, out_spec],
        out_shape=out_shape,
    )(rows)

    return (total[0, 0], point[0, 0], jnp.zeros((), f32), chamfer[0, 0])


# speedup vs baseline: 2.4910x; 1.4887x over previous
import jax
import jax.numpy as jnp
from jax.experimental import pallas as pl
from jax.experimental.pallas import tpu as pltpu

W_POINT, W_CHAMFER, W_STRUCTURE = 1.0, 5.0, 2.0
BIG = 1e10


def _split3_bf16(x):
    hi = x.astype(jnp.bfloat16)
    r1 = x - hi.astype(jnp.float32)
    mid = r1.astype(jnp.bfloat16)
    lo = (r1 - mid.astype(jnp.float32)).astype(jnp.bfloat16)
    return hi, mid, lo


def _loss_kernel(rows, out_total, out_point, out_chamfer, buf_a, buf_b):
    B = rows.shape[0]
    N = rows.shape[2]
    bf16 = jnp.bfloat16
    f32 = jnp.float32
    one = jnp.ones((1, N), bf16)
    zero3 = jnp.zeros((3, N), bf16)
    dn = (((0,), (0,)), ((), ()))

    def fill(b, buf):
        blk = rows[b]
        px, py, pv = blk[0:1], blk[1:2], blk[2:3]
        tx, ty = blk[3:4], blk[4:5]
        pm = (pv == 1.0).astype(f32)
        a2p = px * px + py * py + (1.0 - pm) * BIG
        ahi, amid, alo = _split3_bf16(a2p)
        p_pen = jnp.concatenate(
            [-2.0 * px.astype(bf16), -2.0 * py.astype(bf16),
             ahi, amid, alo, zero3], axis=0)
        t_one = jnp.concatenate(
            [tx.astype(bf16), ty.astype(bf16), one, one, one, zero3], axis=0)
        buf[...] = jax.lax.dot_general(p_pen, t_one, dn,
                                       preferred_element_type=f32)

    def reduce(b, buf):
        blk = rows[b]
        px, py, pv = blk[0:1], blk[1:2], blk[2:3]
        tx, ty, tv = blk[3:4], blk[4:5], blk[5:6]
        s0, s1 = blk[6:7], blk[7:8]

        v = (tv == 1.0).astype(f32)
        ex = px - tx
        ey = py - ty
        err2 = ex * ex + ey * ey
        point_sse = jnp.sum(v * err2)
        tmask = jnp.clip(s0 + s1, 0.0, 1.0) * v
        struct_sse = jnp.sum(tmask * err2)
        tm_sum = jnp.sum(tmask)

        pm = (pv == 1.0).astype(f32)
        cnt_p = jnp.sum(pm)
        cnt_t = jnp.sum(v)
        a2 = px * px + py * py
        b2 = tx * tx + ty * ty
        b2t = b2 + (1.0 - v) * BIG
        a2p = a2 + (1.0 - pm) * BIG

        m = buf[...]
        tgtmin = jnp.min(m, axis=0, keepdims=True)
        d2t = jnp.maximum(b2 + tgtmin, 0.0) + 1e-12
        mean_t = jnp.sum(v * jnp.sqrt(d2t)) / jnp.maximum(cnt_t, 1.0)

        z = jnp.min(m + b2t, axis=1, keepdims=True)
        predmin = jnp.transpose(z, (1, 0)) - a2p
        d2p = jnp.maximum(a2 + predmin, 0.0) + 1e-12
        mean_p = jnp.sum(pm * jnp.sqrt(d2p)) / jnp.maximum(cnt_p, 1.0)

        valid_b = ((cnt_p > 0.0) & (cnt_t > 0.0)).astype(f32)
        chamfer = valid_b * (mean_p + mean_t) * 0.5
        return point_sse, struct_sse, tm_sum, chamfer

    def add4(c, r):
        return (c[0] + r[0], c[1] + r[1], c[2] + r[2], c[3] + r[3])

    fill(0, buf_a)

    def body(k, carry):
        fill(2 * k + 1, buf_b)
        carry = add4(carry, reduce(2 * k, buf_a))
        fill(2 * k + 2, buf_a)
        carry = add4(carry, reduce(2 * k + 1, buf_b))
        return carry

    z4 = (jnp.float32(0.0),) * 4
    carry = jax.lax.fori_loop(0, (B - 2) // 2, body, z4)
    fill(B - 1, buf_b)
    carry = add4(carry, reduce(B - 2, buf_a))
    pt_acc, st_acc, tm_acc, ch_acc = add4(carry, reduce(B - 1, buf_b))

    n_elems = jnp.float32(B) * jnp.float32(2 * N)
    loss_point = pt_acc / n_elems
    loss_structure = jnp.where(tm_acc == 0.0, 0.0, st_acc / n_elems)
    loss_chamfer = ch_acc / jnp.float32(B)
    out_point[0, 0] = loss_point
    out_chamfer[0, 0] = loss_chamfer
    out_total[0, 0] = (W_POINT * loss_point + W_CHAMFER * loss_chamfer
                       + W_STRUCTURE * loss_structure)


def kernel(pred, target, skeleton_mask):
    B, N, _ = pred.shape
    f32 = jnp.float32

    rows = jnp.stack(
        [pred[:, :, 0], pred[:, :, 1], pred[:, :, 2],
         target[:, :, 0], target[:, :, 1], target[:, :, 2],
         skeleton_mask[:, :, 0].astype(f32),
         skeleton_mask[:, :, 1].astype(f32)], axis=1)

    out_spec = pl.BlockSpec(memory_space=pltpu.SMEM)
    out_shape = [jax.ShapeDtypeStruct((1, 1), f32)] * 3
    total, point, chamfer = pl.pallas_call(
        _loss_kernel,
        out_specs=[out_spec, out_spec, out_spec],
        out_shape=out_shape,
        scratch_shapes=[pltpu.VMEM((N, N), f32), pltpu.VMEM((N, N), f32)],
    )(rows)

    return (total[0, 0], point[0, 0], jnp.zeros((), f32), chamfer[0, 0])


# speedup vs baseline: 2.9499x; 1.1842x over previous
import jax
import jax.numpy as jnp
from jax.experimental import pallas as pl
from jax.experimental.pallas import tpu as pltpu

W_POINT, W_CHAMFER, W_STRUCTURE = 1.0, 5.0, 2.0
BIG = 1e10


def _split3_bf16(x):
    hi = x.astype(jnp.bfloat16)
    r1 = x - hi.astype(jnp.float32)
    mid = r1.astype(jnp.bfloat16)
    lo = (r1 - mid.astype(jnp.float32)).astype(jnp.bfloat16)
    return hi, mid, lo


def _loss_kernel(rows, out_total, out_point, out_chamfer,
                 buf_a, buf_b, buf_c, buf_d):
    B = rows.shape[0]
    N = rows.shape[2]
    bf16 = jnp.bfloat16
    f32 = jnp.float32
    one = jnp.ones((1, N), bf16)
    dn = (((0,), (0,)), ((), ()))

    pxa, pya = rows[:, 0, :], rows[:, 1, :]
    txa, tya, tva = rows[:, 3, :], rows[:, 4, :], rows[:, 5, :]
    s0a, s1a = rows[:, 6, :], rows[:, 7, :]
    va = (tva == 1.0).astype(f32)
    exa = pxa - txa
    eya = pya - tya
    err2 = exa * exa + eya * eya
    pt_acc = jnp.sum(va * err2)
    tmaska = jnp.clip(s0a + s1a, 0.0, 1.0) * va
    st_acc = jnp.sum(tmaska * err2)
    tm_acc = jnp.sum(tmaska)

    def fill(b, buf):
        blk = rows[b]
        px, py, pv = blk[0:1], blk[1:2], blk[2:3]
        tx, ty, tv = blk[3:4], blk[4:5], blk[5:6]
        pm = (pv == 1.0).astype(f32)
        v = (tv == 1.0).astype(f32)
        a2p = px * px + py * py + (1.0 - pm) * BIG
        b2t = tx * tx + ty * ty + (1.0 - v) * BIG
        ahi, amid, alo = _split3_bf16(a2p)
        bhi, bmid, blo = _split3_bf16(b2t)
        p_pen = jnp.concatenate(
            [-2.0 * px.astype(bf16), -2.0 * py.astype(bf16),
             ahi, amid, alo, one, one, one], axis=0)
        t_pen = jnp.concatenate(
            [tx.astype(bf16), ty.astype(bf16), one, one, one,
             bhi, bmid, blo], axis=0)
        buf[...] = jax.lax.dot_general(p_pen, t_pen, dn,
                                       preferred_element_type=f32)

    def reduce(b, buf):
        blk = rows[b]
        pv, tv = blk[2:3], blk[5:6]
        v = (tv == 1.0).astype(f32)
        pm = (pv == 1.0).astype(f32)
        cnt_p = jnp.sum(pm)
        cnt_t = jnp.sum(v)
        pen_p = (1.0 - pm) * BIG
        pen_t = (1.0 - v) * BIG

        m = buf[...]
        tgtmin = jnp.min(m, axis=0, keepdims=True)
        d2t = jnp.maximum(tgtmin - pen_t, 0.0) + 1e-12
        mean_t = jnp.sum(v * jnp.sqrt(d2t)) / jnp.maximum(cnt_t, 1.0)

        z = jnp.min(m, axis=1, keepdims=True)
        d2p = jnp.maximum(jnp.transpose(z, (1, 0)) - pen_p, 0.0) + 1e-12
        mean_p = jnp.sum(pm * jnp.sqrt(d2p)) / jnp.maximum(cnt_p, 1.0)

        valid_b = ((cnt_p > 0.0) & (cnt_t > 0.0)).astype(f32)
        return valid_b * (mean_p + mean_t) * 0.5

    bufs = (buf_a, buf_b, buf_c, buf_d)
    fill(0, buf_a)

    def body(k, ch):
        s = 4 * k
        for c in range(4):
            fill(jnp.minimum(s + c + 1, B - 1), bufs[(c + 1) % 4])
            ch = ch + reduce(s + c, bufs[c])
        return ch

    ch_acc = jax.lax.fori_loop(0, B // 4, body, jnp.float32(0.0))

    n_elems = jnp.float32(B) * jnp.float32(2 * N)
    loss_point = pt_acc / n_elems
    loss_structure = jnp.where(tm_acc == 0.0, 0.0, st_acc / n_elems)
    loss_chamfer = ch_acc / jnp.float32(B)
    out_point[0, 0] = loss_point
    out_chamfer[0, 0] = loss_chamfer
    out_total[0, 0] = (W_POINT * loss_point + W_CHAMFER * loss_chamfer
                       + W_STRUCTURE * loss_structure)


def kernel(pred, target, skeleton_mask):
    B, N, _ = pred.shape
    f32 = jnp.float32

    rows = jnp.stack(
        [pred[:, :, 0], pred[:, :, 1], pred[:, :, 2],
         target[:, :, 0], target[:, :, 1], target[:, :, 2],
         skeleton_mask[:, :, 0].astype(f32),
         skeleton_mask[:, :, 1].astype(f32)], axis=1)

    out_spec = pl.BlockSpec(memory_space=pltpu.SMEM)
    out_shape = [jax.ShapeDtypeStruct((1, 1), f32)] * 3
    total, point, chamfer = pl.pallas_call(
        _loss_kernel,
        out_specs=[out_spec, out_spec, out_spec],
        out_shape=out_shape,
        scratch_shapes=[pltpu.VMEM((N, N), f32)] * 4,
    )(rows)

    return (total[0, 0], point[0, 0], jnp.zeros((), f32), chamfer[0, 0])
